# baseline (speedup 1.0000x reference)
import functools

import jax
import jax.numpy as jnp
from jax.experimental import pallas as pl
from jax.experimental.pallas import tpu as pltpu
from jax.experimental.pallas import tpu_sc as plsc

_NUM_CLUSTERS = 1000
_ROWS = _NUM_CLUSTERS + 1
_EMB = 128
_GW = 128
_BW = 256


def _sc_gather(table, idx_ltb, B, T, L):
    mesh = plsc.VectorSubcoreMesh(core_axis_name="c", subcore_axis_name="s")

    @functools.partial(
        pl.kernel,
        out_type=jax.ShapeDtypeStruct((T, L, B, _EMB), jnp.float32),
        mesh=mesh,
        scratch_types=[pltpu.VMEM_SHARED((L * _ROWS, _EMB), jnp.float32)],
    )
    def k(table_hbm, idx_hbm, out_hbm, table_sh):
        sid = jax.lax.axis_index("s")

        @pl.when(sid == 0)
        def _():
            pltpu.sync_copy(table_hbm, table_sh)

        plsc.subcore_barrier()

        def body(i_vmem, o_vmem):
            for j in range(_BW // _GW):
                pltpu.sync_copy(
                    table_sh.at[i_vmem.at[0, 0, pl.ds(j * _GW, _GW)]],
                    o_vmem.at[0, 0, pl.ds(j * _GW, _GW)],
                )

        nbw = B // _BW
        pltpu.emit_pipeline(
            body,
            grid=(T * L * nbw,),
            in_specs=[
                pl.BlockSpec(
                    (1, 1, _BW),
                    index_map=lambda i: ((i // nbw) % L, i // (L * nbw), i % nbw),
                )
            ],
            out_specs=[
                pl.BlockSpec(
                    (1, 1, _BW, _EMB),
                    index_map=lambda i: (i // (L * nbw), (i // nbw) % L, i % nbw, 0),
                )
            ],
            core_axis_name=("c", "s"),
            dimension_semantics=(pltpu.PARALLEL,),
        )(idx_hbm, out_hbm)

    return k(table, idx_ltb)


def kernel(x, W0, W1, W2):
    B, T, L = x.shape
    table = jnp.concatenate([W0, W1, W2], axis=0)
    offs = (jnp.arange(L, dtype=jnp.int32) * _ROWS).astype(x.dtype)
    idx_ltb = jnp.transpose(x + offs, (2, 1, 0))
    out = _sc_gather(table, idx_ltb, B, T, L)
    return jnp.transpose(out, (2, 0, 1, 3))


# speedup vs baseline: 1.0283x; 1.0283x over previous
import functools

import jax
import jax.numpy as jnp
from jax.experimental import pallas as pl
from jax.experimental.pallas import tpu as pltpu
from jax.experimental.pallas import tpu_sc as plsc

_NUM_CLUSTERS = 1000
_ROWS = _NUM_CLUSTERS + 1
_BANK = 1008
_EMB = 128
_GW = 128


def _sc_gather(tables, idx_ltb, B, T, L):
    mesh = plsc.VectorSubcoreMesh(core_axis_name="c", subcore_axis_name="s")
    nb = B // _GW

    @functools.partial(
        pl.kernel,
        out_type=jax.ShapeDtypeStruct((T, L, B, _EMB), jnp.float32),
        mesh=mesh,
        scratch_types=[pltpu.VMEM_SHARED((L * _BANK, _EMB), jnp.float32)],
    )
    def k(*refs):
        w_hbms = refs[:L]
        idx_hbm, out_hbm, table_sh = refs[L], refs[L + 1], refs[L + 2]
        sid = jax.lax.axis_index("s")

        @pl.when(sid == 0)
        def _():
            for l in range(L):
                pltpu.sync_copy(w_hbms[l], table_sh.at[pl.ds(l * _BANK, _ROWS)])

        plsc.subcore_barrier()

        def body(i_vmem, o_vmem):
            pltpu.sync_copy(table_sh.at[i_vmem.at[0, 0]], o_vmem.at[0, 0])

        pltpu.emit_pipeline(
            body,
            grid=(T * L * nb,),
            in_specs=[
                pl.BlockSpec(
                    (1, 1, _GW),
                    index_map=lambda i: ((i // nb) % L, i // (L * nb), i % nb),
                )
            ],
            out_specs=[
                pl.BlockSpec(
                    (1, 1, _GW, _EMB),
                    index_map=lambda i: (i // (L * nb), (i // nb) % L, i % nb, 0),
                )
            ],
            core_axis_name=("c", "s"),
            dimension_semantics=(pltpu.PARALLEL,),
        )(idx_hbm, out_hbm)

    return k(*tables, idx_ltb)


def kernel(x, W0, W1, W2):
    B, T, L = x.shape
    offs = (jnp.arange(L, dtype=jnp.int32) * _BANK).astype(x.dtype)
    idx_ltb = jnp.transpose(x + offs, (2, 1, 0))
    out = _sc_gather((W0, W1, W2), idx_ltb, B, T, L)
    return jnp.transpose(out, (2, 0, 1, 3))


# speedup vs baseline: 1.0300x; 1.0016x over previous
import functools

import jax
import jax.numpy as jnp
from jax.experimental import pallas as pl
from jax.experimental.pallas import tpu as pltpu
from jax.experimental.pallas import tpu_sc as plsc

_NUM_CLUSTERS = 1000
_ROWS = _NUM_CLUSTERS + 1
_EMB = 128
_GW = 128


def _sc_gather(tables, idx_ltb, B, T, L):
    mesh = plsc.VectorSubcoreMesh(core_axis_name="c", subcore_axis_name="s")
    nb = B // _GW

    @functools.partial(
        pl.kernel,
        out_type=jax.ShapeDtypeStruct((T, L, B, _EMB), jnp.float32),
        mesh=mesh,
        scratch_types=[pltpu.VMEM_SHARED((_ROWS, _EMB), jnp.float32)] * L,
    )
    def k(*refs):
        w_hbms = refs[:L]
        idx_hbm, out_hbm = refs[L], refs[L + 1]
        tables_sh = refs[L + 2:]
        sid = jax.lax.axis_index("s")

        @pl.when(sid == 0)
        def _():
            for l in range(L):
                pltpu.sync_copy(w_hbms[l], tables_sh[l])

        plsc.subcore_barrier()

        for l in range(L):
            table_sh = tables_sh[l]

            def body(i_vmem, o_vmem, table_sh=table_sh):
                pltpu.sync_copy(table_sh.at[i_vmem.at[0, 0]], o_vmem.at[0, 0])

            pltpu.emit_pipeline(
                body,
                grid=(T * nb,),
                in_specs=[
                    pl.BlockSpec(
                        (1, 1, _GW),
                        index_map=lambda i, l=l: (l, i // nb, i % nb),
                    )
                ],
                out_specs=[
                    pl.BlockSpec(
                        (1, 1, _GW, _EMB),
                        index_map=lambda i, l=l: (i // nb, l, i % nb, 0),
                    )
                ],
                core_axis_name=("c", "s"),
                dimension_semantics=(pltpu.PARALLEL,),
            )(idx_hbm, out_hbm)

    return k(*tables, idx_ltb)


def kernel(x, W0, W1, W2):
    B, T, L = x.shape
    idx_ltb = jnp.transpose(x, (2, 1, 0))
    out = _sc_gather((W0, W1, W2), idx_ltb, B, T, L)
    return jnp.transpose(out, (2, 0, 1, 3))
